# baseline (speedup 1.0000x reference)
import functools

import jax
import jax.numpy as jnp
from jax import lax
from jax.experimental import pallas as pl
from jax.experimental.pallas import tpu as pltpu
from jax.experimental.pallas import tpu_sc as plsc

B, C, HW = 16, 64, 1024
K = 8192
SUB = 512
NSUB = K // SUB
N = B * HW


def _s1_body(x_ref, e_ref, ind_ref, cb_ref):
    b = pl.program_id(0)
    xb = x_ref[0]

    @pl.when(b == 0)
    def _():
        cb_ref[...] = jnp.transpose(e_ref[...])

    eye = (lax.broadcasted_iota(jnp.int32, (C, C), 0)
           == lax.broadcasted_iota(jnp.int32, (C, C), 1)).astype(jnp.float32)
    xt = lax.dot_general(xb, eye, (((0,), (0,)), ((), ())),
                         precision=lax.Precision.HIGHEST,
                         preferred_element_type=jnp.float32)
    x2c = jnp.sum(xt * xt, axis=1, keepdims=True)
    x2 = jnp.transpose(x2c)

    minv = None
    mind = None
    for j in range(NSUB):
        et = e_ref[:, j * SUB:(j + 1) * SUB]
        mm = lax.dot_general(et, xb, (((0,), (0,)), ((), ())),
                             preferred_element_type=jnp.float32)
        e2r = jnp.sum(et * et, axis=0, keepdims=True)
        e2 = jnp.transpose(e2r)
        d = (x2 - 2.0 * mm) + e2
        tmin = jnp.min(d, axis=0, keepdims=True)
        iota = lax.broadcasted_iota(jnp.int32, (SUB, HW), 0)
        targ = (jnp.min(jnp.where(d == tmin, iota, jnp.int32(2**30)),
                        axis=0, keepdims=True) + j * SUB)
        if j == 0:
            minv, mind = tmin, targ
        else:
            upd = tmin < minv
            minv = jnp.where(upd, tmin, minv)
            mind = jnp.where(upd, targ, mind)
    ind_ref[0] = mind


def _argmin_call(x_r, emb):
    return pl.pallas_call(
        _s1_body,
        grid=(B,),
        in_specs=[
            pl.BlockSpec((1, C, HW), lambda b: (b, 0, 0)),
            pl.BlockSpec((C, K), lambda b: (0, 0)),
        ],
        out_specs=[
            pl.BlockSpec((1, 1, HW), lambda b: (b, 0, 0)),
            pl.BlockSpec((K, C), lambda b: (0, 0)),
        ],
        out_shape=[
            jax.ShapeDtypeStruct((B, 1, HW), jnp.int32),
            jax.ShapeDtypeStruct((K, C), jnp.float32),
        ],
    )(x_r, emb)


_NW = 32
_BPW = N // _NW
_CH = _BPW // 128


def _make_gather():
    mesh = plsc.VectorSubcoreMesh(core_axis_name="c", subcore_axis_name="s")

    @functools.partial(
        pl.kernel,
        mesh=mesh,
        compiler_params=pltpu.CompilerParams(use_tc_tiling_on_sc=False),
        out_type=jax.ShapeDtypeStruct((N, C), jnp.float32),
        scratch_types=[
            pltpu.VMEM((_CH, 128), jnp.int32),
            pltpu.VMEM((_CH, 128, C), jnp.float32),
            pltpu.SemaphoreType.DMA,
        ],
    )
    def gather_k(table_hbm, idx_hbm, out_hbm, idx_v, rows_v, sem):
        wid = lax.axis_index("s") * 2 + lax.axis_index("c")
        base = wid * _BPW
        pltpu.sync_copy(idx_hbm.at[pl.ds(wid * _CH, _CH)], idx_v)
        copies = [
            pltpu.async_copy(table_hbm.at[idx_v.at[j]], rows_v.at[j], sem)
            for j in range(_CH)
        ]
        for cp in copies:
            cp.wait()
        for j in range(_CH):
            pltpu.sync_copy(rows_v.at[j],
                            out_hbm.at[pl.ds(base + j * 128, 128)])

    return gather_k


def _s3_body(q_ref, x_ref, out_ref, diff_ref):
    qb = q_ref[0]
    xb = x_ref[0]
    qt = jnp.transpose(qb)
    st = qt - xb
    out_ref[0] = xb + st
    p = jnp.sum(st * st)
    diff_ref[0] = jnp.full((1, 128), p, dtype=jnp.float32)


def _finish_call(q, x_r):
    return pl.pallas_call(
        _s3_body,
        grid=(B,),
        in_specs=[
            pl.BlockSpec((1, HW, C), lambda b: (b, 0, 0)),
            pl.BlockSpec((1, C, HW), lambda b: (b, 0, 0)),
        ],
        out_specs=[
            pl.BlockSpec((1, C, HW), lambda b: (b, 0, 0)),
            pl.BlockSpec((1, 1, 128), lambda b: (b, 0, 0)),
        ],
        out_shape=[
            jax.ShapeDtypeStruct((B, C, HW), jnp.float32),
            jax.ShapeDtypeStruct((B, 1, 128), jnp.float32),
        ],
    )(q, x_r)


def kernel(input, embedding):
    x_r = input.reshape(B, C, HW)
    ind, codebook = _argmin_call(x_r, embedding)
    idx = ind.reshape(_NW * _CH, 128)
    q = _make_gather()(codebook, idx)
    quant, diffp = _finish_call(q.reshape(B, HW, C), x_r)
    diff = jnp.sum(diffp[:, 0, 0]) / jnp.float32(N * C)
    return (quant.reshape(B, C, 32, 32), diff,
            ind.reshape(B, 32, 32))
